# baseline (speedup 1.0000x reference)
import jax
import jax.numpy as jnp
from jax import lax
from jax.experimental import pallas as pl
from jax.experimental.pallas import tpu as pltpu
from jax.experimental.pallas import tpu_sc as plsc

B = 1024
K1 = 512
D = 128
N = 1000000
INV_T = 1.0 / 0.07
M = 0.5

NC = 2
NS = 16
NW = NC * NS
BPW = B // NW
CH = 64
NCHUNK = K1 // CH

_f32 = jnp.float32
_i32 = jnp.int32


_CBLK = 8000


def _copy_body(src1, src2, dst1, dst2):
    dst1[...] = src1[...]
    dst2[...] = src2[...]


_copy_banks = pl.pallas_call(
    _copy_body,
    grid=(N // _CBLK,),
    in_specs=[pl.BlockSpec((_CBLK, D), lambda i: (i, 0))] * 2,
    out_specs=[pl.BlockSpec((_CBLK, D), lambda i: (i, 0))] * 2,
    out_shape=(
        jax.ShapeDtypeStruct((N, D), _f32),
        jax.ShapeDtypeStruct((N, D), _f32),
    ),
    name="bank_copy",
)


def _rsqrt16(s):
    i = plsc.bitcast(s, _i32)
    i = jnp.int32(0x5F3759DF) - (i >> 1)
    r = plsc.bitcast(i, _f32)
    for _ in range(3):
        r = r * (1.5 - 0.5 * s * r * r)
    return r


def _sc_dot_body(l_hbm, ab_hbm, idx_hbm, meml_hbm, memab_hbm,
                 outl_hbm, outab_hbm,
                 idx2_v, rl0, rl1, rl2, rl3, rab0, rab1, rab2, rab3,
                 actl2_v, actab2_v, outl_v, outab_v, tbuf_a, tbuf_b,
                 sl0, sl1, sl2, sl3, sab0, sab1, sab2, sab3, sem_i):
    cid = lax.axis_index("c")
    sid = lax.axis_index("s")
    wid = sid * NC + cid
    b0 = wid * BPW

    iota16 = lax.iota(_i32, 16)

    ring_l = (rl0, rl1, rl2, rl3)
    ring_ab = (rab0, rab1, rab2, rab3)
    sems_l = (sl0, sl1, sl2, sl3)
    sems_ab = (sab0, sab1, sab2, sab3)

    pltpu.sync_copy(idx_hbm.at[b0], idx2_v.at[0])
    pltpu.sync_copy(l_hbm.at[b0], actl2_v.at[0])
    pltpu.sync_copy(ab_hbm.at[b0], actab2_v.at[0])

    def bbody(j, _):
        jp = j & 1
        jq = 1 - jp
        b = b0 + j
        bn = b0 + jnp.minimum(j + 1, BPW - 1)

        def fire(c):
            isl = idx2_v.at[jp, pl.ds(c * CH, CH)]
            s = c % 4
            return (
                pltpu.async_copy(meml_hbm.at[isl], ring_l[s], sems_l[s]),
                pltpu.async_copy(memab_hbm.at[isl], ring_ab[s], sems_ab[s]),
            )

        cps = {c: fire(c) for c in range(4)}
        cpi = pltpu.async_copy(idx_hbm.at[bn], idx2_v.at[jq], sem_i)
        cpl2 = pltpu.async_copy(l_hbm.at[bn], actl2_v.at[jq], sem_i)
        cpab2 = pltpu.async_copy(ab_hbm.at[bn], actab2_v.at[jq], sem_i)
        vls = [actl2_v[jp, pl.ds(t * 16, 16)] for t in range(8)]
        vabs = [actab2_v[jp, pl.ds(t * 16, 16)] for t in range(8)]
        for c in range(NCHUNK):
            cp_l, cp_ab = cps.pop(c)
            cp_l.wait()
            cp_ab.wait()
            rl = ring_l[c % 4]
            rab = ring_ab[c % 4]

            def kkbody(kk, _2, rl=rl, rab=rab, coff=c * CH):
                for rows, vch, outv, tb in (
                    (rl, vabs, outab_v, tbuf_a),
                    (rab, vls, outl_v, tbuf_b),
                ):
                    for kj in range(16):
                        r = kk * 16 + kj
                        prods = [rows[r, pl.ds(t * 16, 16)] * vch[t] for t in range(8)]
                        while len(prods) > 1:
                            prods = [prods[i] + prods[i + 1]
                                     for i in range(0, len(prods), 2)]
                        tb[kj, pl.ds(0, 16)] = prods[0]
                    cols = [plsc.load_gather(tb, [iota16, jnp.full((16,), i, _i32)])
                            for i in range(16)]
                    while len(cols) > 1:
                        cols = [cols[i] + cols[i + 1] for i in range(0, len(cols), 2)]
                    outv[pl.ds(coff + kk * 16, 16)] = cols[0] * INV_T
                return 0

            lax.fori_loop(0, CH // 16, kkbody, 0)
            if c + 4 < NCHUNK:
                cps[c + 4] = fire(c + 4)
        pltpu.sync_copy(outl_v, outl_hbm.at[b])
        pltpu.sync_copy(outab_v, outab_hbm.at[b])
        cpi.wait()
        cpl2.wait()
        cpab2.wait()
        return 0

    lax.fori_loop(0, BPW, bbody, 0)


_sc_dot = pl.kernel(
    _sc_dot_body,
    out_type=(
        jax.ShapeDtypeStruct((B, K1), _f32),
        jax.ShapeDtypeStruct((B, K1), _f32),
    ),
    mesh=plsc.VectorSubcoreMesh(
        core_axis_name="c", subcore_axis_name="s", num_cores=NC, num_subcores=NS
    ),
    compiler_params=pltpu.CompilerParams(needs_layout_passes=False),
    scratch_types=(
        [pltpu.VMEM((2, K1), _i32)]
        + [pltpu.VMEM((CH, D), _f32)] * 8
        + [
            pltpu.VMEM((2, D), _f32),
            pltpu.VMEM((2, D), _f32),
            pltpu.VMEM((K1,), _f32),
            pltpu.VMEM((K1,), _f32),
            pltpu.VMEM((16, 17), _f32),
            pltpu.VMEM((16, 17), _f32),
        ]
        + [pltpu.SemaphoreType.DMA] * 9
    ),
    name="cmcmem_sc",
)


def _sc_scatter_body(l_hbm, ab_hbm, y_hbm, meml_hbm, memab_hbm,
                     newl_ref, newab_ref,
                     ys_v, old_v, act32_v, upd_v, sem_u):
    cid = lax.axis_index("c")
    sid = lax.axis_index("s")
    wid = sid * NC + cid
    b0 = wid * BPW
    pltpu.sync_copy(y_hbm.at[pl.ds(b0, BPW)], ys_v)
    iota16 = lax.iota(_i32, 16)

    def _update_bank(mem_hbm, act_hbm, new_ref):
        pltpu.async_copy(mem_hbm.at[ys_v], old_v, sem_u).wait()
        pltpu.sync_copy(act_hbm.at[pl.ds(b0, BPW)], act32_v)
        for half in range(BPW // 16):
            rids = iota16 + half * 16

            def p1(d, ssq):
                cd = jnp.full((16,), d, _i32)
                oc = plsc.load_gather(old_v, [rids, cd])
                ac = plsc.load_gather(act32_v, [rids, cd])
                pos = oc * M + ac * (1.0 - M)
                plsc.store_scatter(upd_v, [rids, cd], pos)
                return ssq + pos * pos

            ssq = lax.fori_loop(0, D, p1, jnp.zeros((16,), _f32))
            r = _rsqrt16(ssq)

            def p2(d, _):
                cd = jnp.full((16,), d, _i32)
                pc = plsc.load_gather(upd_v, [rids, cd])
                plsc.store_scatter(upd_v, [rids, cd], pc * r)
                return 0

            lax.fori_loop(0, D, p2, 0)
        pltpu.async_copy(upd_v, new_ref.at[ys_v], sem_u).wait()

    _update_bank(meml_hbm, l_hbm, newl_ref)
    _update_bank(memab_hbm, ab_hbm, newab_ref)


_sc_scatter = pl.kernel(
    _sc_scatter_body,
    out_type=(),
    mesh=plsc.VectorSubcoreMesh(
        core_axis_name="c", subcore_axis_name="s", num_cores=NC, num_subcores=NS
    ),
    compiler_params=pltpu.CompilerParams(needs_layout_passes=False),
    scratch_types=[
        pltpu.VMEM((BPW,), _i32),
        pltpu.VMEM((BPW, D), _f32),
        pltpu.VMEM((BPW, D), _f32),
        pltpu.VMEM((BPW, D), _f32),
        pltpu.SemaphoreType.DMA,
    ],
    name="cmcmem_sc_scatter",
)


def kernel(l, ab, y, idx, memory_l, memory_ab):
    out_l, out_ab = _sc_dot(l, ab, idx, memory_l, memory_ab)
    new_l, new_ab = _copy_banks(memory_l, memory_ab)
    nl_ref = jax.new_ref(new_l)
    nab_ref = jax.new_ref(new_ab)
    _sc_scatter(l, ab, y, memory_l, memory_ab, nl_ref, nab_ref)
    return out_l, out_ab, nl_ref[...], nab_ref[...]


# speedup vs baseline: 1.6499x; 1.6499x over previous
import jax
import jax.numpy as jnp
from jax import lax
from jax.experimental import pallas as pl
from jax.experimental.pallas import tpu as pltpu
from jax.experimental.pallas import tpu_sc as plsc

B = 1024
K1 = 512
D = 128
N = 1000000
INV_T = 1.0 / 0.07
M = 0.5

NC = 2
NS = 16
NW = NC * NS
BPW = B // NW
CH = 64
NCHUNK = K1 // CH

_f32 = jnp.float32
_i32 = jnp.int32


_CBLK = 8000


def _copy_body(src1, src2, dst1, dst2):
    dst1[...] = src1[...]
    dst2[...] = src2[...]


_copy_banks = pl.pallas_call(
    _copy_body,
    grid=(N // _CBLK,),
    in_specs=[pl.BlockSpec((_CBLK, D), lambda i: (i, 0))] * 2,
    out_specs=[pl.BlockSpec((_CBLK, D), lambda i: (i, 0))] * 2,
    out_shape=(
        jax.ShapeDtypeStruct((N, D), _f32),
        jax.ShapeDtypeStruct((N, D), _f32),
    ),
    name="bank_copy",
)


def _rsqrt16(s):
    i = plsc.bitcast(s, _i32)
    i = jnp.int32(0x5F3759DF) - (i >> 1)
    r = plsc.bitcast(i, _f32)
    for _ in range(3):
        r = r * (1.5 - 0.5 * s * r * r)
    return r


def _sc_dot_body(l_hbm, ab_hbm, y_hbm, idx_hbm, meml_hbm, memab_hbm,
                 outl_hbm, outab_hbm, updl_hbm, updab_hbm,
                 idx2_v, rl0, rl1, rl2, rl3, rab0, rab1, rab2, rab3,
                 actl2_v, actab2_v, outl_v, outab_v, tbuf_a, tbuf_b,
                 ys_v, old_v, act32_v, upd_v,
                 sl0, sl1, sl2, sl3, sab0, sab1, sab2, sab3, sem_i, sem_u):
    cid = lax.axis_index("c")
    sid = lax.axis_index("s")
    wid = sid * NC + cid
    b0 = wid * BPW

    iota16 = lax.iota(_i32, 16)

    pltpu.sync_copy(y_hbm.at[pl.ds(b0, BPW)], ys_v)

    def _update_bank(mem_hbm, act_hbm, upd_hbm):
        pltpu.async_copy(mem_hbm.at[ys_v], old_v, sem_u).wait()
        pltpu.sync_copy(act_hbm.at[pl.ds(b0, BPW)], act32_v)
        for half in range(BPW // 16):
            rids = iota16 + half * 16

            def p1(d, ssq):
                cd = jnp.full((16,), d, _i32)
                oc = plsc.load_gather(old_v, [rids, cd])
                ac = plsc.load_gather(act32_v, [rids, cd])
                pos = oc * M + ac * (1.0 - M)
                plsc.store_scatter(upd_v, [rids, cd], pos)
                return ssq + pos * pos

            ssq = lax.fori_loop(0, D, p1, jnp.zeros((16,), _f32))
            r = _rsqrt16(ssq)

            def p2(d, _):
                cd = jnp.full((16,), d, _i32)
                pc = plsc.load_gather(upd_v, [rids, cd])
                plsc.store_scatter(upd_v, [rids, cd], pc * r)
                return 0

            lax.fori_loop(0, D, p2, 0)
        pltpu.sync_copy(upd_v, upd_hbm.at[pl.ds(b0, BPW)])

    _update_bank(meml_hbm, l_hbm, updl_hbm)
    _update_bank(memab_hbm, ab_hbm, updab_hbm)

    ring_l = (rl0, rl1, rl2, rl3)
    ring_ab = (rab0, rab1, rab2, rab3)
    sems_l = (sl0, sl1, sl2, sl3)
    sems_ab = (sab0, sab1, sab2, sab3)

    pltpu.sync_copy(idx_hbm.at[b0], idx2_v.at[0])
    pltpu.sync_copy(l_hbm.at[b0], actl2_v.at[0])
    pltpu.sync_copy(ab_hbm.at[b0], actab2_v.at[0])

    def bbody(j, _):
        jp = j & 1
        jq = 1 - jp
        b = b0 + j
        bn = b0 + jnp.minimum(j + 1, BPW - 1)

        def fire(c):
            isl = idx2_v.at[jp, pl.ds(c * CH, CH)]
            s = c % 4
            return (
                pltpu.async_copy(meml_hbm.at[isl], ring_l[s], sems_l[s]),
                pltpu.async_copy(memab_hbm.at[isl], ring_ab[s], sems_ab[s]),
            )

        cps = {c: fire(c) for c in range(4)}
        cpi = pltpu.async_copy(idx_hbm.at[bn], idx2_v.at[jq], sem_i)
        cpl2 = pltpu.async_copy(l_hbm.at[bn], actl2_v.at[jq], sem_i)
        cpab2 = pltpu.async_copy(ab_hbm.at[bn], actab2_v.at[jq], sem_i)
        vls = [actl2_v[jp, pl.ds(t * 16, 16)] for t in range(8)]
        vabs = [actab2_v[jp, pl.ds(t * 16, 16)] for t in range(8)]
        for c in range(NCHUNK):
            cp_l, cp_ab = cps.pop(c)
            cp_l.wait()
            cp_ab.wait()
            rl = ring_l[c % 4]
            rab = ring_ab[c % 4]

            def kkbody(kk, _2, rl=rl, rab=rab, coff=c * CH):
                for rows, vch, outv, tb in (
                    (rl, vabs, outab_v, tbuf_a),
                    (rab, vls, outl_v, tbuf_b),
                ):
                    for kj in range(16):
                        r = kk * 16 + kj
                        prods = [rows[r, pl.ds(t * 16, 16)] * vch[t] for t in range(8)]
                        while len(prods) > 1:
                            prods = [prods[i] + prods[i + 1]
                                     for i in range(0, len(prods), 2)]
                        tb[kj, pl.ds(0, 16)] = prods[0]
                    cols = [plsc.load_gather(tb, [iota16, jnp.full((16,), i, _i32)])
                            for i in range(16)]
                    while len(cols) > 1:
                        cols = [cols[i] + cols[i + 1] for i in range(0, len(cols), 2)]
                    outv[pl.ds(coff + kk * 16, 16)] = cols[0] * INV_T
                return 0

            lax.fori_loop(0, CH // 16, kkbody, 0)
            if c + 4 < NCHUNK:
                cps[c + 4] = fire(c + 4)
        pltpu.sync_copy(outl_v, outl_hbm.at[b])
        pltpu.sync_copy(outab_v, outab_hbm.at[b])
        cpi.wait()
        cpl2.wait()
        cpab2.wait()
        return 0

    lax.fori_loop(0, BPW, bbody, 0)


_sc_dot = pl.kernel(
    _sc_dot_body,
    out_type=(
        jax.ShapeDtypeStruct((B, K1), _f32),
        jax.ShapeDtypeStruct((B, K1), _f32),
        jax.ShapeDtypeStruct((B, D), _f32),
        jax.ShapeDtypeStruct((B, D), _f32),
    ),
    mesh=plsc.VectorSubcoreMesh(
        core_axis_name="c", subcore_axis_name="s", num_cores=NC, num_subcores=NS
    ),
    compiler_params=pltpu.CompilerParams(needs_layout_passes=False),
    scratch_types=(
        [pltpu.VMEM((2, K1), _i32)]
        + [pltpu.VMEM((CH, D), _f32)] * 8
        + [
            pltpu.VMEM((2, D), _f32),
            pltpu.VMEM((2, D), _f32),
            pltpu.VMEM((K1,), _f32),
            pltpu.VMEM((K1,), _f32),
            pltpu.VMEM((16, 17), _f32),
            pltpu.VMEM((16, 17), _f32),
            pltpu.VMEM((BPW,), _i32),
            pltpu.VMEM((BPW, D), _f32),
            pltpu.VMEM((BPW, D), _f32),
            pltpu.VMEM((BPW, D), _f32),
        ]
        + [pltpu.SemaphoreType.DMA] * 10
    ),
    name="cmcmem_sc",
)


def _sc_scatter_body(y_hbm, updl_hbm, updab_hbm, newl_ref, newab_ref,
                     ys_v, rows_v, sem_a):
    cid = lax.axis_index("c")
    sid = lax.axis_index("s")
    wid = sid * NC + cid
    b0 = wid * BPW
    pltpu.sync_copy(y_hbm.at[pl.ds(b0, BPW)], ys_v)
    pltpu.sync_copy(updl_hbm.at[pl.ds(b0, BPW)], rows_v)
    pltpu.async_copy(rows_v, newl_ref.at[ys_v], sem_a).wait()
    pltpu.sync_copy(updab_hbm.at[pl.ds(b0, BPW)], rows_v)
    pltpu.async_copy(rows_v, newab_ref.at[ys_v], sem_a).wait()


_sc_scatter = pl.kernel(
    _sc_scatter_body,
    out_type=(),
    mesh=plsc.VectorSubcoreMesh(
        core_axis_name="c", subcore_axis_name="s", num_cores=NC, num_subcores=NS
    ),
    compiler_params=pltpu.CompilerParams(needs_layout_passes=False),
    scratch_types=[
        pltpu.VMEM((BPW,), _i32),
        pltpu.VMEM((BPW, D), _f32),
        pltpu.SemaphoreType.DMA,
    ],
    name="cmcmem_sc_scatter",
)


def kernel(l, ab, y, idx, memory_l, memory_ab):
    out_l, out_ab, upd_l, upd_ab = _sc_dot(l, ab, y, idx, memory_l, memory_ab)
    new_l, new_ab = _copy_banks(memory_l, memory_ab)
    nl_ref = jax.new_ref(new_l)
    nab_ref = jax.new_ref(new_ab)
    _sc_scatter(y, upd_l, upd_ab, nl_ref, nab_ref)
    return out_l, out_ab, nl_ref[...], nab_ref[...]


# speedup vs baseline: 1.7256x; 1.0459x over previous
import jax
import jax.numpy as jnp
from jax import lax
from jax.experimental import pallas as pl
from jax.experimental.pallas import tpu as pltpu
from jax.experimental.pallas import tpu_sc as plsc

B = 1024
K1 = 512
D = 128
N = 1000000
INV_T = 1.0 / 0.07
M = 0.5

NC = 2
NS = 16
NW = NC * NS
BPW = B // NW
CH = 64
NCHUNK = K1 // CH

_f32 = jnp.float32
_i32 = jnp.int32


_CBLK = 8000


def _copy_body(src1, src2, dst1, dst2):
    dst1[...] = src1[...]
    dst2[...] = src2[...]


_copy_banks = pl.pallas_call(
    _copy_body,
    grid=(N // _CBLK,),
    in_specs=[pl.BlockSpec((_CBLK, D), lambda i: (i, 0))] * 2,
    out_specs=[pl.BlockSpec((_CBLK, D), lambda i: (i, 0))] * 2,
    out_shape=(
        jax.ShapeDtypeStruct((N, D), _f32),
        jax.ShapeDtypeStruct((N, D), _f32),
    ),
    name="bank_copy",
)


def _rsqrt16(s):
    i = plsc.bitcast(s, _i32)
    i = jnp.int32(0x5F3759DF) - (i >> 1)
    r = plsc.bitcast(i, _f32)
    for _ in range(3):
        r = r * (1.5 - 0.5 * s * r * r)
    return r


def _sc_dot_body(l_hbm, ab_hbm, y_hbm, idx_hbm, meml_hbm, memab_hbm,
                 outl_hbm, outab_hbm, updl_hbm, updab_hbm,
                 idx2_v, rl0, rl1, rl2, rl3, rab0, rab1, rab2, rab3,
                 actl2_v, actab2_v, outl_v, outab_v, tbuf_a, tbuf_b,
                 ys_v, old_v, act32_v, upd_v,
                 sl0, sl1, sl2, sl3, sab0, sab1, sab2, sab3, sem_i, sem_u):
    cid = lax.axis_index("c")
    sid = lax.axis_index("s")
    wid = sid * NC + cid
    b0 = wid * BPW

    iota16 = lax.iota(_i32, 16)

    pltpu.sync_copy(y_hbm.at[pl.ds(b0, BPW)], ys_v)

    def _update_bank(mem_hbm, act_hbm, upd_hbm):
        pltpu.async_copy(mem_hbm.at[ys_v], old_v, sem_u).wait()
        pltpu.sync_copy(act_hbm.at[pl.ds(b0, BPW)], act32_v)
        for half in range(BPW // 16):
            rids = iota16 + half * 16

            def p1(d, ssq):
                cd = jnp.full((16,), d, _i32)
                oc = plsc.load_gather(old_v, [rids, cd])
                ac = plsc.load_gather(act32_v, [rids, cd])
                pos = oc * M + ac * (1.0 - M)
                plsc.store_scatter(upd_v, [rids, cd], pos)
                return ssq + pos * pos

            ssq = lax.fori_loop(0, D, p1, jnp.zeros((16,), _f32))
            r = _rsqrt16(ssq)

            def p2(d, _):
                cd = jnp.full((16,), d, _i32)
                pc = plsc.load_gather(upd_v, [rids, cd])
                plsc.store_scatter(upd_v, [rids, cd], pc * r)
                return 0

            lax.fori_loop(0, D, p2, 0)
        pltpu.sync_copy(upd_v, upd_hbm.at[pl.ds(b0, BPW)])

    _update_bank(meml_hbm, l_hbm, updl_hbm)
    _update_bank(memab_hbm, ab_hbm, updab_hbm)

    ring_l = (rl0, rl1, rl2, rl3)
    ring_ab = (rab0, rab1, rab2, rab3)
    sems_l = (sl0, sl1, sl2, sl3)
    sems_ab = (sab0, sab1, sab2, sab3)

    def fire(jslot, c):
        isl = idx2_v.at[jslot, pl.ds(c * CH, CH)]
        s = c % 4
        pltpu.async_copy(meml_hbm.at[isl], ring_l[s], sems_l[s])
        pltpu.async_copy(memab_hbm.at[isl], ring_ab[s], sems_ab[s])

    pltpu.sync_copy(idx_hbm.at[b0], idx2_v.at[0])
    pltpu.sync_copy(l_hbm.at[b0], actl2_v.at[0])
    pltpu.sync_copy(ab_hbm.at[b0], actab2_v.at[0])
    for c in range(4):
        fire(0, c)

    def bbody(j, _):
        jp = j & 1
        jq = 1 - jp
        b = b0 + j
        bn = b0 + jnp.minimum(j + 1, BPW - 1)

        cpi = pltpu.async_copy(idx_hbm.at[bn], idx2_v.at[jq], sem_i)
        cpl2 = pltpu.async_copy(l_hbm.at[bn], actl2_v.at[jq], sem_i)
        cpab2 = pltpu.async_copy(ab_hbm.at[bn], actab2_v.at[jq], sem_i)
        vls = [actl2_v[jp, pl.ds(t * 16, 16)] for t in range(8)]
        vabs = [actab2_v[jp, pl.ds(t * 16, 16)] for t in range(8)]
        for c in range(NCHUNK):
            s = c % 4
            isl = idx2_v.at[jp, pl.ds(c * CH, CH)]
            pltpu.make_async_copy(meml_hbm.at[isl], ring_l[s], sems_l[s]).wait()
            pltpu.make_async_copy(memab_hbm.at[isl], ring_ab[s], sems_ab[s]).wait()
            rl = ring_l[c % 4]
            rab = ring_ab[c % 4]

            def kkbody(kk, _2, rl=rl, rab=rab, coff=c * CH):
                for rows, vch, outv, tb in (
                    (rl, vabs, outab_v, tbuf_a),
                    (rab, vls, outl_v, tbuf_b),
                ):
                    for kj in range(16):
                        r = kk * 16 + kj
                        prods = [rows[r, pl.ds(t * 16, 16)] * vch[t] for t in range(8)]
                        while len(prods) > 1:
                            prods = [prods[i] + prods[i + 1]
                                     for i in range(0, len(prods), 2)]
                        tb[kj, pl.ds(0, 16)] = prods[0]
                    cols = [plsc.load_gather(tb, [iota16, jnp.full((16,), i, _i32)])
                            for i in range(16)]
                    while len(cols) > 1:
                        cols = [cols[i] + cols[i + 1] for i in range(0, len(cols), 2)]
                    outv[pl.ds(coff + kk * 16, 16)] = cols[0] * INV_T
                return 0

            lax.fori_loop(0, CH // 16, kkbody, 0)
            if c + 4 < NCHUNK:
                fire(jp, c + 4)
            else:
                if c == 4:
                    cpi.wait()
                    cpl2.wait()
                    cpab2.wait()

                @pl.when(j < BPW - 1)
                def _fire_next(c=c):
                    fire(jq, c - 4)
        pltpu.sync_copy(outl_v, outl_hbm.at[b])
        pltpu.sync_copy(outab_v, outab_hbm.at[b])
        return 0

    lax.fori_loop(0, BPW, bbody, 0)


_sc_dot = pl.kernel(
    _sc_dot_body,
    out_type=(
        jax.ShapeDtypeStruct((B, K1), _f32),
        jax.ShapeDtypeStruct((B, K1), _f32),
        jax.ShapeDtypeStruct((B, D), _f32),
        jax.ShapeDtypeStruct((B, D), _f32),
    ),
    mesh=plsc.VectorSubcoreMesh(
        core_axis_name="c", subcore_axis_name="s", num_cores=NC, num_subcores=NS
    ),
    compiler_params=pltpu.CompilerParams(needs_layout_passes=False),
    scratch_types=(
        [pltpu.VMEM((2, K1), _i32)]
        + [pltpu.VMEM((CH, D), _f32)] * 8
        + [
            pltpu.VMEM((2, D), _f32),
            pltpu.VMEM((2, D), _f32),
            pltpu.VMEM((K1,), _f32),
            pltpu.VMEM((K1,), _f32),
            pltpu.VMEM((16, 17), _f32),
            pltpu.VMEM((16, 17), _f32),
            pltpu.VMEM((BPW,), _i32),
            pltpu.VMEM((BPW, D), _f32),
            pltpu.VMEM((BPW, D), _f32),
            pltpu.VMEM((BPW, D), _f32),
        ]
        + [pltpu.SemaphoreType.DMA] * 10
    ),
    name="cmcmem_sc",
)


def _sc_scatter_body(y_hbm, updl_hbm, updab_hbm, newl_ref, newab_ref,
                     ys_v, rows_v, sem_a):
    cid = lax.axis_index("c")
    sid = lax.axis_index("s")
    wid = sid * NC + cid
    b0 = wid * BPW
    pltpu.sync_copy(y_hbm.at[pl.ds(b0, BPW)], ys_v)
    pltpu.sync_copy(updl_hbm.at[pl.ds(b0, BPW)], rows_v)
    pltpu.async_copy(rows_v, newl_ref.at[ys_v], sem_a).wait()
    pltpu.sync_copy(updab_hbm.at[pl.ds(b0, BPW)], rows_v)
    pltpu.async_copy(rows_v, newab_ref.at[ys_v], sem_a).wait()


_sc_scatter = pl.kernel(
    _sc_scatter_body,
    out_type=(),
    mesh=plsc.VectorSubcoreMesh(
        core_axis_name="c", subcore_axis_name="s", num_cores=NC, num_subcores=NS
    ),
    compiler_params=pltpu.CompilerParams(needs_layout_passes=False),
    scratch_types=[
        pltpu.VMEM((BPW,), _i32),
        pltpu.VMEM((BPW, D), _f32),
        pltpu.SemaphoreType.DMA,
    ],
    name="cmcmem_sc_scatter",
)


def kernel(l, ab, y, idx, memory_l, memory_ab):
    out_l, out_ab, upd_l, upd_ab = _sc_dot(l, ab, y, idx, memory_l, memory_ab)
    new_l, new_ab = _copy_banks(memory_l, memory_ab)
    nl_ref = jax.new_ref(new_l)
    nab_ref = jax.new_ref(new_ab)
    _sc_scatter(y, upd_l, upd_ab, nl_ref, nab_ref)
    return out_l, out_ab, nl_ref[...], nab_ref[...]


# speedup vs baseline: 1.7263x; 1.0004x over previous
import jax
import jax.numpy as jnp
from jax import lax
from jax.experimental import pallas as pl
from jax.experimental.pallas import tpu as pltpu
from jax.experimental.pallas import tpu_sc as plsc

B = 1024
K1 = 512
D = 128
N = 1000000
INV_T = 1.0 / 0.07
M = 0.5

NC = 2
NS = 16
NW = NC * NS
BPW = B // NW
CH = 64
NCHUNK = K1 // CH

_f32 = jnp.float32
_i32 = jnp.int32


_CBLK = 8000


def _copy_body(src1, src2, dst1, dst2):
    dst1[...] = src1[...]
    dst2[...] = src2[...]


_copy_banks = pl.pallas_call(
    _copy_body,
    grid=(N // _CBLK,),
    in_specs=[pl.BlockSpec((_CBLK, D), lambda i: (i, 0))] * 2,
    out_specs=[pl.BlockSpec((_CBLK, D), lambda i: (i, 0))] * 2,
    out_shape=(
        jax.ShapeDtypeStruct((N, D), _f32),
        jax.ShapeDtypeStruct((N, D), _f32),
    ),
    name="bank_copy",
)


def _rsqrt16(s):
    i = plsc.bitcast(s, _i32)
    i = jnp.int32(0x5F3759DF) - (i >> 1)
    r = plsc.bitcast(i, _f32)
    for _ in range(3):
        r = r * (1.5 - 0.5 * s * r * r)
    return r


def _sc_dot_body(l_hbm, ab_hbm, y_hbm, idx_hbm, meml_hbm, memab_hbm,
                 outl_hbm, outab_hbm, updl_hbm, updab_hbm,
                 idx2_v, rl0, rl1, rl2, rl3, rab0, rab1, rab2, rab3,
                 actl2_v, actab2_v, outl_v, outab_v, tbuf_a, tbuf_b,
                 ys_v, old_v, act32_v, upd_v,
                 sl0, sl1, sl2, sl3, sab0, sab1, sab2, sab3,
                 sem_i, sem_u, sem_o):
    cid = lax.axis_index("c")
    sid = lax.axis_index("s")
    wid = sid * NC + cid
    b0 = wid * BPW

    iota16 = lax.iota(_i32, 16)

    ring_l = (rl0, rl1, rl2, rl3)
    ring_ab = (rab0, rab1, rab2, rab3)
    sems_l = (sl0, sl1, sl2, sl3)
    sems_ab = (sab0, sab1, sab2, sab3)

    def fire(jslot, c):
        isl = idx2_v.at[jslot, pl.ds(c * CH, CH)]
        s = c % 4
        pltpu.async_copy(meml_hbm.at[isl], ring_l[s], sems_l[s])
        pltpu.async_copy(memab_hbm.at[isl], ring_ab[s], sems_ab[s])

    pltpu.sync_copy(idx_hbm.at[b0], idx2_v.at[0])
    pltpu.sync_copy(l_hbm.at[b0], actl2_v.at[0])
    pltpu.sync_copy(ab_hbm.at[b0], actab2_v.at[0])
    for c in range(4):
        fire(0, c)

    pltpu.sync_copy(y_hbm.at[pl.ds(b0, BPW)], ys_v)

    def _update_bank(mem_hbm, act_hbm, upd_hbm):
        pltpu.async_copy(mem_hbm.at[ys_v], old_v, sem_u).wait()
        pltpu.sync_copy(act_hbm.at[pl.ds(b0, BPW)], act32_v)
        for half in range(BPW // 16):
            rids = iota16 + half * 16

            def p1(d, ssq):
                cd = jnp.full((16,), d, _i32)
                oc = plsc.load_gather(old_v, [rids, cd])
                ac = plsc.load_gather(act32_v, [rids, cd])
                pos = oc * M + ac * (1.0 - M)
                plsc.store_scatter(upd_v, [rids, cd], pos)
                return ssq + pos * pos

            ssq = lax.fori_loop(0, D, p1, jnp.zeros((16,), _f32))
            r = _rsqrt16(ssq)

            def p2(d, _):
                cd = jnp.full((16,), d, _i32)
                pc = plsc.load_gather(upd_v, [rids, cd])
                plsc.store_scatter(upd_v, [rids, cd], pc * r)
                return 0

            lax.fori_loop(0, D, p2, 0)
        pltpu.sync_copy(upd_v, upd_hbm.at[pl.ds(b0, BPW)])

    _update_bank(meml_hbm, l_hbm, updl_hbm)
    _update_bank(memab_hbm, ab_hbm, updab_hbm)

    def bbody(j, _):
        jp = j & 1
        jq = 1 - jp
        b = b0 + j
        bn = b0 + jnp.minimum(j + 1, BPW - 1)

        cpi = pltpu.async_copy(idx_hbm.at[bn], idx2_v.at[jq], sem_i)
        cpl2 = pltpu.async_copy(l_hbm.at[bn], actl2_v.at[jq], sem_i)
        cpab2 = pltpu.async_copy(ab_hbm.at[bn], actab2_v.at[jq], sem_i)
        vls = [actl2_v[jp, pl.ds(t * 16, 16)] for t in range(8)]
        vabs = [actab2_v[jp, pl.ds(t * 16, 16)] for t in range(8)]
        for c in range(NCHUNK):
            s = c % 4
            isl = idx2_v.at[jp, pl.ds(c * CH, CH)]
            pltpu.make_async_copy(meml_hbm.at[isl], ring_l[s], sems_l[s]).wait()
            pltpu.make_async_copy(memab_hbm.at[isl], ring_ab[s], sems_ab[s]).wait()
            rl = ring_l[c % 4]
            rab = ring_ab[c % 4]

            def kkbody(kk, _2, rl=rl, rab=rab, coff=c * CH):
                for rows, vch, outv, tb in (
                    (rl, vabs, outab_v, tbuf_a),
                    (rab, vls, outl_v, tbuf_b),
                ):
                    for kj in range(16):
                        r = kk * 16 + kj
                        prods = [rows[r, pl.ds(t * 16, 16)] * vch[t] for t in range(8)]
                        while len(prods) > 1:
                            prods = [prods[i] + prods[i + 1]
                                     for i in range(0, len(prods), 2)]
                        tb[kj, pl.ds(0, 16)] = prods[0]
                    cols = [plsc.load_gather(tb, [iota16, jnp.full((16,), i, _i32)])
                            for i in range(16)]
                    while len(cols) > 1:
                        cols = [cols[i] + cols[i + 1] for i in range(0, len(cols), 2)]
                    outv[jp, pl.ds(coff + kk * 16, 16)] = cols[0] * INV_T
                return 0

            lax.fori_loop(0, CH // 16, kkbody, 0)
            if c + 4 < NCHUNK:
                fire(jp, c + 4)
                if c == 0:
                    @pl.when(j > 0)
                    def _drain_out():
                        pltpu.make_async_copy(
                            outl_v.at[jq], outl_hbm.at[b], sem_o).wait()
                        pltpu.make_async_copy(
                            outab_v.at[jq], outab_hbm.at[b], sem_o).wait()
            else:
                if c == 4:
                    cpi.wait()
                    cpl2.wait()
                    cpab2.wait()

                @pl.when(j < BPW - 1)
                def _fire_next(c=c):
                    fire(jq, c - 4)
        pltpu.async_copy(outl_v.at[jp], outl_hbm.at[b], sem_o)
        pltpu.async_copy(outab_v.at[jp], outab_hbm.at[b], sem_o)
        return 0

    lax.fori_loop(0, BPW, bbody, 0)
    blast = b0 + BPW - 1
    pltpu.make_async_copy(outl_v.at[1], outl_hbm.at[blast], sem_o).wait()
    pltpu.make_async_copy(outab_v.at[1], outab_hbm.at[blast], sem_o).wait()


_sc_dot = pl.kernel(
    _sc_dot_body,
    out_type=(
        jax.ShapeDtypeStruct((B, K1), _f32),
        jax.ShapeDtypeStruct((B, K1), _f32),
        jax.ShapeDtypeStruct((B, D), _f32),
        jax.ShapeDtypeStruct((B, D), _f32),
    ),
    mesh=plsc.VectorSubcoreMesh(
        core_axis_name="c", subcore_axis_name="s", num_cores=NC, num_subcores=NS
    ),
    compiler_params=pltpu.CompilerParams(needs_layout_passes=False),
    scratch_types=(
        [pltpu.VMEM((2, K1), _i32)]
        + [pltpu.VMEM((CH, D), _f32)] * 8
        + [
            pltpu.VMEM((2, D), _f32),
            pltpu.VMEM((2, D), _f32),
            pltpu.VMEM((2, K1), _f32),
            pltpu.VMEM((2, K1), _f32),
            pltpu.VMEM((16, 17), _f32),
            pltpu.VMEM((16, 17), _f32),
            pltpu.VMEM((BPW,), _i32),
            pltpu.VMEM((BPW, D), _f32),
            pltpu.VMEM((BPW, D), _f32),
            pltpu.VMEM((BPW, D), _f32),
        ]
        + [pltpu.SemaphoreType.DMA] * 11
    ),
    name="cmcmem_sc",
)


def _sc_scatter_body(y_hbm, updl_hbm, updab_hbm, newl_ref, newab_ref,
                     ys_v, rows_v, sem_a):
    cid = lax.axis_index("c")
    sid = lax.axis_index("s")
    wid = sid * NC + cid
    b0 = wid * BPW
    pltpu.sync_copy(y_hbm.at[pl.ds(b0, BPW)], ys_v)
    pltpu.sync_copy(updl_hbm.at[pl.ds(b0, BPW)], rows_v)
    pltpu.async_copy(rows_v, newl_ref.at[ys_v], sem_a).wait()
    pltpu.sync_copy(updab_hbm.at[pl.ds(b0, BPW)], rows_v)
    pltpu.async_copy(rows_v, newab_ref.at[ys_v], sem_a).wait()


_sc_scatter = pl.kernel(
    _sc_scatter_body,
    out_type=(),
    mesh=plsc.VectorSubcoreMesh(
        core_axis_name="c", subcore_axis_name="s", num_cores=NC, num_subcores=NS
    ),
    compiler_params=pltpu.CompilerParams(needs_layout_passes=False),
    scratch_types=[
        pltpu.VMEM((BPW,), _i32),
        pltpu.VMEM((BPW, D), _f32),
        pltpu.SemaphoreType.DMA,
    ],
    name="cmcmem_sc_scatter",
)


def kernel(l, ab, y, idx, memory_l, memory_ab):
    out_l, out_ab, upd_l, upd_ab = _sc_dot(l, ab, y, idx, memory_l, memory_ab)
    new_l, new_ab = _copy_banks(memory_l, memory_ab)
    nl_ref = jax.new_ref(new_l)
    nab_ref = jax.new_ref(new_ab)
    _sc_scatter(y, upd_l, upd_ab, nl_ref, nab_ref)
    return out_l, out_ab, nl_ref[...], nab_ref[...]


# speedup vs baseline: 1.7266x; 1.0002x over previous
import jax
import jax.numpy as jnp
from jax import lax
from jax.experimental import pallas as pl
from jax.experimental.pallas import tpu as pltpu
from jax.experimental.pallas import tpu_sc as plsc

B = 1024
K1 = 512
D = 128
N = 1000000
INV_T = 1.0 / 0.07
M = 0.5

NC = 2
NS = 16
NW = NC * NS
BPW = B // NW
CH = 128
NCHUNK = K1 // CH
NRING = 2

_f32 = jnp.float32
_i32 = jnp.int32


_CBLK = 8000


def _copy_body(src1, src2, dst1, dst2):
    dst1[...] = src1[...]
    dst2[...] = src2[...]


_copy_banks = pl.pallas_call(
    _copy_body,
    grid=(N // _CBLK,),
    in_specs=[pl.BlockSpec((_CBLK, D), lambda i: (i, 0))] * 2,
    out_specs=[pl.BlockSpec((_CBLK, D), lambda i: (i, 0))] * 2,
    out_shape=(
        jax.ShapeDtypeStruct((N, D), _f32),
        jax.ShapeDtypeStruct((N, D), _f32),
    ),
    name="bank_copy",
)


def _rsqrt16(s):
    i = plsc.bitcast(s, _i32)
    i = jnp.int32(0x5F3759DF) - (i >> 1)
    r = plsc.bitcast(i, _f32)
    for _ in range(3):
        r = r * (1.5 - 0.5 * s * r * r)
    return r


def _sc_dot_body(l_hbm, ab_hbm, y_hbm, idx_hbm, meml_hbm, memab_hbm,
                 outl_hbm, outab_hbm, updl_hbm, updab_hbm,
                 idx2_v, rl0, rl1, rab0, rab1,
                 actl2_v, actab2_v, outl_v, outab_v, tbuf_a, tbuf_b,
                 ys_v, old_v, act32_v, upd_v,
                 sl0, sl1, sab0, sab1,
                 sem_i, sem_u, sem_o):
    cid = lax.axis_index("c")
    sid = lax.axis_index("s")
    wid = sid * NC + cid
    b0 = wid * BPW

    iota16 = lax.iota(_i32, 16)

    ring_l = (rl0, rl1)
    ring_ab = (rab0, rab1)
    sems_l = (sl0, sl1)
    sems_ab = (sab0, sab1)

    def fire(jslot, c):
        isl = idx2_v.at[jslot, pl.ds(c * CH, CH)]
        s = c % NRING
        pltpu.async_copy(meml_hbm.at[isl], ring_l[s], sems_l[s])
        pltpu.async_copy(memab_hbm.at[isl], ring_ab[s], sems_ab[s])

    pltpu.sync_copy(idx_hbm.at[b0], idx2_v.at[0])
    pltpu.sync_copy(l_hbm.at[b0], actl2_v.at[0])
    pltpu.sync_copy(ab_hbm.at[b0], actab2_v.at[0])
    for c in range(NRING):
        fire(0, c)

    pltpu.sync_copy(y_hbm.at[pl.ds(b0, BPW)], ys_v)

    def _update_bank(mem_hbm, act_hbm, upd_hbm):
        pltpu.async_copy(mem_hbm.at[ys_v], old_v, sem_u).wait()
        pltpu.sync_copy(act_hbm.at[pl.ds(b0, BPW)], act32_v)
        for half in range(BPW // 16):
            rids = iota16 + half * 16

            def p1(d, ssq):
                cd = jnp.full((16,), d, _i32)
                oc = plsc.load_gather(old_v, [rids, cd])
                ac = plsc.load_gather(act32_v, [rids, cd])
                pos = oc * M + ac * (1.0 - M)
                plsc.store_scatter(upd_v, [rids, cd], pos)
                return ssq + pos * pos

            ssq = lax.fori_loop(0, D, p1, jnp.zeros((16,), _f32))
            r = _rsqrt16(ssq)

            def p2(d, _):
                cd = jnp.full((16,), d, _i32)
                pc = plsc.load_gather(upd_v, [rids, cd])
                plsc.store_scatter(upd_v, [rids, cd], pc * r)
                return 0

            lax.fori_loop(0, D, p2, 0)
        pltpu.sync_copy(upd_v, upd_hbm.at[pl.ds(b0, BPW)])

    _update_bank(meml_hbm, l_hbm, updl_hbm)
    _update_bank(memab_hbm, ab_hbm, updab_hbm)

    def bbody(j, _):
        jp = j & 1
        jq = 1 - jp
        b = b0 + j
        bn = b0 + jnp.minimum(j + 1, BPW - 1)

        cpi = pltpu.async_copy(idx_hbm.at[bn], idx2_v.at[jq], sem_i)
        cpl2 = pltpu.async_copy(l_hbm.at[bn], actl2_v.at[jq], sem_i)
        cpab2 = pltpu.async_copy(ab_hbm.at[bn], actab2_v.at[jq], sem_i)
        vls = [actl2_v[jp, pl.ds(t * 16, 16)] for t in range(8)]
        vabs = [actab2_v[jp, pl.ds(t * 16, 16)] for t in range(8)]
        for c in range(NCHUNK):
            s = c % NRING
            isl = idx2_v.at[jp, pl.ds(c * CH, CH)]
            pltpu.make_async_copy(meml_hbm.at[isl], ring_l[s], sems_l[s]).wait()
            pltpu.make_async_copy(memab_hbm.at[isl], ring_ab[s], sems_ab[s]).wait()
            rl = ring_l[s]
            rab = ring_ab[s]

            def kkbody(kk, _2, rl=rl, rab=rab, coff=c * CH):
                for rows, vch, outv, tb in (
                    (rl, vabs, outab_v, tbuf_a),
                    (rab, vls, outl_v, tbuf_b),
                ):
                    for kj in range(16):
                        r = kk * 16 + kj
                        prods = [rows[r, pl.ds(t * 16, 16)] * vch[t] for t in range(8)]
                        while len(prods) > 1:
                            prods = [prods[i] + prods[i + 1]
                                     for i in range(0, len(prods), 2)]
                        tb[kj, pl.ds(0, 16)] = prods[0]
                    cols = [plsc.load_gather(tb, [iota16, jnp.full((16,), i, _i32)])
                            for i in range(16)]
                    while len(cols) > 1:
                        cols = [cols[i] + cols[i + 1] for i in range(0, len(cols), 2)]
                    outv[jp, pl.ds(coff + kk * 16, 16)] = cols[0] * INV_T
                return 0

            lax.fori_loop(0, CH // 16, kkbody, 0)
            if c + NRING < NCHUNK:
                fire(jp, c + NRING)
                if c == 0:
                    @pl.when(j > 0)
                    def _drain_out():
                        pltpu.make_async_copy(
                            outl_v.at[jq], outl_hbm.at[b], sem_o).wait()
                        pltpu.make_async_copy(
                            outab_v.at[jq], outab_hbm.at[b], sem_o).wait()
            else:
                if c == NCHUNK - NRING:
                    cpi.wait()
                    cpl2.wait()
                    cpab2.wait()

                @pl.when(j < BPW - 1)
                def _fire_next(c=c):
                    fire(jq, c - (NCHUNK - NRING))
        pltpu.async_copy(outl_v.at[jp], outl_hbm.at[b], sem_o)
        pltpu.async_copy(outab_v.at[jp], outab_hbm.at[b], sem_o)
        return 0

    lax.fori_loop(0, BPW, bbody, 0)
    blast = b0 + BPW - 1
    pltpu.make_async_copy(outl_v.at[1], outl_hbm.at[blast], sem_o).wait()
    pltpu.make_async_copy(outab_v.at[1], outab_hbm.at[blast], sem_o).wait()


_sc_dot = pl.kernel(
    _sc_dot_body,
    out_type=(
        jax.ShapeDtypeStruct((B, K1), _f32),
        jax.ShapeDtypeStruct((B, K1), _f32),
        jax.ShapeDtypeStruct((B, D), _f32),
        jax.ShapeDtypeStruct((B, D), _f32),
    ),
    mesh=plsc.VectorSubcoreMesh(
        core_axis_name="c", subcore_axis_name="s", num_cores=NC, num_subcores=NS
    ),
    compiler_params=pltpu.CompilerParams(needs_layout_passes=False),
    scratch_types=(
        [pltpu.VMEM((2, K1), _i32)]
        + [pltpu.VMEM((CH, D), _f32)] * 4
        + [
            pltpu.VMEM((2, D), _f32),
            pltpu.VMEM((2, D), _f32),
            pltpu.VMEM((2, K1), _f32),
            pltpu.VMEM((2, K1), _f32),
            pltpu.VMEM((16, 17), _f32),
            pltpu.VMEM((16, 17), _f32),
            pltpu.VMEM((BPW,), _i32),
            pltpu.VMEM((BPW, D), _f32),
            pltpu.VMEM((BPW, D), _f32),
            pltpu.VMEM((BPW, D), _f32),
        ]
        + [pltpu.SemaphoreType.DMA] * 7
    ),
    name="cmcmem_sc",
)


def _sc_scatter_body(y_hbm, updl_hbm, updab_hbm, newl_ref, newab_ref,
                     ys_v, rows_v, sem_a):
    cid = lax.axis_index("c")
    sid = lax.axis_index("s")
    wid = sid * NC + cid
    b0 = wid * BPW
    pltpu.sync_copy(y_hbm.at[pl.ds(b0, BPW)], ys_v)
    pltpu.sync_copy(updl_hbm.at[pl.ds(b0, BPW)], rows_v)
    pltpu.async_copy(rows_v, newl_ref.at[ys_v], sem_a).wait()
    pltpu.sync_copy(updab_hbm.at[pl.ds(b0, BPW)], rows_v)
    pltpu.async_copy(rows_v, newab_ref.at[ys_v], sem_a).wait()


_sc_scatter = pl.kernel(
    _sc_scatter_body,
    out_type=(),
    mesh=plsc.VectorSubcoreMesh(
        core_axis_name="c", subcore_axis_name="s", num_cores=NC, num_subcores=NS
    ),
    compiler_params=pltpu.CompilerParams(needs_layout_passes=False),
    scratch_types=[
        pltpu.VMEM((BPW,), _i32),
        pltpu.VMEM((BPW, D), _f32),
        pltpu.SemaphoreType.DMA,
    ],
    name="cmcmem_sc_scatter",
)


def kernel(l, ab, y, idx, memory_l, memory_ab):
    out_l, out_ab, upd_l, upd_ab = _sc_dot(l, ab, y, idx, memory_l, memory_ab)
    new_l, new_ab = _copy_banks(memory_l, memory_ab)
    nl_ref = jax.new_ref(new_l)
    nab_ref = jax.new_ref(new_ab)
    _sc_scatter(y, upd_l, upd_ab, nl_ref, nab_ref)
    return out_l, out_ab, nl_ref[...], nab_ref[...]
